# baseline (speedup 1.0000x reference)
import functools

import jax
import jax.numpy as jnp
from jax import lax
from jax.experimental import pallas as pl
from jax.experimental.pallas import tpu as pltpu
from jax.experimental.pallas import tpu_sc as plsc

VOCAB = 1_000_000
D = 64
L = 128
NC, NS = 2, 16
NW = NC * NS

R = 4


def _gather_kernel(nrows_per_worker, nchunks, idx_hbm, table_hbm, out_hbm,
                   idx_v, rows_v, sem):
    wid = lax.axis_index("s") * NC + lax.axis_index("c")
    row_base = wid * nrows_per_worker

    def body(g, carry):
        row0 = row_base + g * R
        pltpu.sync_copy(idx_hbm.at[pl.ds(row0, R)], idx_v)
        cps = [
            pltpu.async_copy(
                table_hbm.at[idx_v.at[j]],
                rows_v.at[pl.ds(j * L, L)],
                sem,
            )
            for j in range(R)
        ]
        for c in cps:
            c.wait()
        pltpu.sync_copy(rows_v, out_hbm.at[pl.ds(row0 * L, R * L)])
        return carry

    lax.fori_loop(0, nchunks, body, 0, unroll=False)


def kernel(inputs, weight):
    B = inputs.shape[0] * inputs.shape[1]
    nrows = B // L
    nrows_per_worker = nrows // NW
    nchunks = nrows_per_worker // R

    idx2d = inputs.reshape(nrows, L).astype(jnp.int32)

    mesh = plsc.VectorSubcoreMesh(core_axis_name="c", subcore_axis_name="s")
    k = pl.kernel(
        functools.partial(_gather_kernel, nrows_per_worker, nchunks),
        out_type=jax.ShapeDtypeStruct((B, D), jnp.float32),
        mesh=mesh,
        scratch_types=[
            pltpu.VMEM((R, L), jnp.int32),
            pltpu.VMEM((R * L, D), jnp.float32),
            pltpu.SemaphoreType.DMA,
        ],
        compiler_params=pltpu.CompilerParams(use_tc_tiling_on_sc=False),
    )
    out = k(idx2d, weight)
    return out.reshape(inputs.shape[0], inputs.shape[1], D)


# speedup vs baseline: 1.0434x; 1.0434x over previous
import functools

import jax
import jax.numpy as jnp
from jax import lax
from jax.experimental import pallas as pl
from jax.experimental.pallas import tpu as pltpu
from jax.experimental.pallas import tpu_sc as plsc

D = 64
L = 128
NC, NS = 2, 16
NW = NC * NS

R = 5


def _gather_kernel(nrw, nchunks, idx_hbm, table_hbm, out_hbm,
                   idx_all, rows0, rows1, gs0, gs1, ws0, ws1):
    wid = lax.axis_index("s") * NC + lax.axis_index("c")
    row_base = wid * nrw
    rows = (rows0, rows1)
    g_sem = (gs0, gs1)
    w_sem = (ws0, ws1)

    pltpu.sync_copy(idx_hbm.at[pl.ds(row_base, nrw)], idx_all)

    def fire_gather(i, b):
        for j in range(R):
            pltpu.async_copy(
                table_hbm.at[idx_all.at[i * R + j]],
                rows[b].at[pl.ds(j * L, L)],
                g_sem[b],
            )

    def drain_gather(b):
        for j in range(R):
            pltpu.make_async_copy(
                table_hbm.at[idx_all.at[j]],
                rows[b].at[pl.ds(j * L, L)],
                g_sem[b],
            ).wait()

    def fire_wb(i, b):
        pltpu.async_copy(
            rows[b],
            out_hbm.at[pl.ds((row_base + i * R) * L, R * L)],
            w_sem[b],
        )

    def drain_wb(b):
        pltpu.make_async_copy(
            rows[b],
            out_hbm.at[pl.ds(0, R * L)],
            w_sem[b],
        ).wait()

    fire_gather(0, 0)
    fire_gather(1, 1)

    def pair_body(p, carry):
        for b in (0, 1):
            i = 2 * p + b
            drain_gather(b)
            fire_wb(i, b)
            drain_wb(b)
            fire_gather(i + 2, b)
        return carry

    lax.fori_loop(0, nchunks // 2 - 1, pair_body, 0, unroll=False)

    for b in (0, 1):
        drain_gather(b)
        fire_wb(nchunks - 2 + b, b)
    for b in (0, 1):
        drain_wb(b)


def kernel(inputs, weight):
    B = inputs.shape[0] * inputs.shape[1]
    nrows = B // L
    nrw = nrows // NW
    nchunks = nrw // R

    idx2d = inputs.reshape(nrows, L).astype(jnp.int32)

    mesh = plsc.VectorSubcoreMesh(core_axis_name="c", subcore_axis_name="s")
    k = pl.kernel(
        functools.partial(_gather_kernel, nrw, nchunks),
        out_type=jax.ShapeDtypeStruct((B, D), jnp.float32),
        mesh=mesh,
        scratch_types=[
            pltpu.VMEM((nrw, L), jnp.int32),
            pltpu.VMEM((R * L, D), jnp.float32),
            pltpu.VMEM((R * L, D), jnp.float32),
            pltpu.SemaphoreType.DMA,
            pltpu.SemaphoreType.DMA,
            pltpu.SemaphoreType.DMA,
            pltpu.SemaphoreType.DMA,
        ],
        compiler_params=pltpu.CompilerParams(use_tc_tiling_on_sc=False),
    )
    out = k(idx2d, weight)
    return out.reshape(inputs.shape[0], inputs.shape[1], D)
